# Initial kernel scaffold; baseline (speedup 1.0000x reference)
import jax
import jax.numpy as jnp
from jax.experimental import pallas as pl


def kernel(inp, table):
    raise NotImplementedError("write your pallas kernel here")


# speedup vs baseline: 1.7972x; 1.7972x over previous
import functools

import jax
import jax.numpy as jnp
from jax import lax
from jax.experimental import pallas as pl
from jax.experimental.pallas import tpu as pltpu
from jax.experimental.pallas import tpu_sc as plsc

_D = 64
_BATCH = 16384
_HIST = 50
_B_TOTAL = _BATCH * _HIST
_NC = 2
_NS = 16
_NW = _NC * _NS
_B_PER_W = _B_TOTAL // _NW
_IDXW = 128
_K = 4
_CHUNK = _K * _IDXW
_N_CHUNKS = _B_PER_W // _CHUNK


def _make_gather(mesh):
    @functools.partial(
        pl.kernel,
        mesh=mesh,
        out_type=jax.ShapeDtypeStruct((_B_TOTAL, _D), jnp.float32),
        compiler_params=pltpu.CompilerParams(use_tc_tiling_on_sc=False),
        scratch_types=[
            pltpu.VMEM((_K, _IDXW), jnp.int32),
            pltpu.VMEM((_CHUNK, _D), jnp.float32),
            pltpu.SemaphoreType.DMA,
        ],
    )
    def gather(idx_hbm, table_hbm, out_hbm, idx_v, rows_v, sem):
        wid = lax.axis_index("s") * _NC + lax.axis_index("c")
        base = wid * _B_PER_W
        idx_row0 = wid * (_B_PER_W // _IDXW)

        def body(c, carry):
            row0 = base + c * _CHUNK
            pltpu.sync_copy(
                idx_hbm.at[pl.ds(idx_row0 + c * _K, _K)], idx_v)
            copies = []
            for k in range(_K):
                copies.append(
                    pltpu.async_copy(
                        table_hbm.at[idx_v.at[k]],
                        rows_v.at[pl.ds(k * _IDXW, _IDXW)],
                        sem,
                    ))
            for cp in copies:
                cp.wait()
            pltpu.sync_copy(rows_v, out_hbm.at[pl.ds(row0, _CHUNK)])
            return carry

        lax.fori_loop(0, _N_CHUNKS, body, 0)

    return gather


def kernel(inp, table):
    mesh = plsc.VectorSubcoreMesh(core_axis_name="c", subcore_axis_name="s")
    idx2d = inp.reshape(_B_TOTAL // _IDXW, _IDXW).astype(jnp.int32)
    out = _make_gather(mesh)(idx2d, table)
    return out.reshape(_BATCH, _HIST, _D)


# speedup vs baseline: 1.8674x; 1.0391x over previous
import functools

import jax
import jax.numpy as jnp
from jax import lax
from jax.experimental import pallas as pl
from jax.experimental.pallas import tpu as pltpu
from jax.experimental.pallas import tpu_sc as plsc

_D = 64
_BATCH = 16384
_HIST = 50
_B_TOTAL = _BATCH * _HIST
_NC = 2
_NS = 16
_NW = _NC * _NS
_B_PER_W = _B_TOTAL // _NW
_IDXW = 128
_K = 4
_CHUNK = _K * _IDXW
_N_CHUNKS = _B_PER_W // _CHUNK
_N_PAIRS = _N_CHUNKS // 2
_IDX_ROWS = _B_PER_W // _IDXW


def _make_gather(mesh):
    @functools.partial(
        pl.kernel,
        mesh=mesh,
        out_type=jax.ShapeDtypeStruct((_B_TOTAL, _D), jnp.float32),
        compiler_params=pltpu.CompilerParams(use_tc_tiling_on_sc=False),
        scratch_types=[
            pltpu.VMEM((_IDX_ROWS, _IDXW), jnp.int32),
            pltpu.VMEM((2, _CHUNK, _D), jnp.float32),
            pltpu.SemaphoreType.DMA,
            pltpu.SemaphoreType.DMA,
            pltpu.SemaphoreType.DMA,
            pltpu.SemaphoreType.DMA,
        ],
    )
    def gather(idx_hbm, table_hbm, out_hbm, idx_v, rows_v, g0, g1, w0, w1):
        wid = lax.axis_index("s") * _NC + lax.axis_index("c")
        base = wid * _B_PER_W
        pltpu.sync_copy(idx_hbm.at[pl.ds(wid * _IDX_ROWS, _IDX_ROWS)], idx_v)

        def fire(c, slot, sem):
            for k in range(_K):
                pltpu.async_copy(
                    table_hbm.at[idx_v.at[c * _K + k]],
                    rows_v.at[slot].at[pl.ds(k * _IDXW, _IDXW)],
                    sem,
                )

        def gwait(c, slot, sem):
            for k in range(_K):
                pltpu.make_async_copy(
                    table_hbm.at[idx_v.at[c * _K + k]],
                    rows_v.at[slot].at[pl.ds(k * _IDXW, _IDXW)],
                    sem,
                ).wait()

        def wstart(c, slot, sem):
            pltpu.async_copy(
                rows_v.at[slot], out_hbm.at[pl.ds(base + c * _CHUNK, _CHUNK)],
                sem)

        def wwait(c, slot, sem):
            pltpu.make_async_copy(
                rows_v.at[slot], out_hbm.at[pl.ds(base + c * _CHUNK, _CHUNK)],
                sem).wait()

        @pl.loop(0, _N_PAIRS)
        def pair(p):
            c0 = p * 2
            c1 = c0 + 1

            @pl.when(p != 0)
            def _():
                wwait(c0 - 2, 0, w0)

            fire(c0, 0, g0)

            @pl.when(p != 0)
            def _():
                wwait(c1 - 2, 1, w1)

            fire(c1, 1, g1)
            gwait(c0, 0, g0)
            wstart(c0, 0, w0)
            gwait(c1, 1, g1)
            wstart(c1, 1, w1)

        wwait(_N_CHUNKS - 2, 0, w0)
        wwait(_N_CHUNKS - 1, 1, w1)

    return gather


def kernel(inp, table):
    mesh = plsc.VectorSubcoreMesh(core_axis_name="c", subcore_axis_name="s")
    idx2d = inp.reshape(_B_TOTAL // _IDXW, _IDXW).astype(jnp.int32)
    out = _make_gather(mesh)(idx2d, table)
    return out.reshape(_BATCH, _HIST, _D)
